# baseline (speedup 1.0000x reference)
import functools

import jax
import jax.numpy as jnp
from jax import lax
from jax.experimental import pallas as pl
from jax.experimental.pallas import tpu as pltpu
from jax.experimental.pallas import tpu_sc as plsc

N = 10000
E = 320000
N_GRAPHS = 64
EPS = 1e-7

NC = 2
NS = 16
EP = E // NS

NBKT = 40
BR = 256
NSLOT = 3
CHB = 128
SLOTCAP = 9600
CAPAL = SLOTCAP + 192
EAP = NSLOT * NS * CAPAL
TEB = 4896
NPAD = NBKT * BR
SCH = 8000
NSCH = E // SCH


def _sc_bucket_body(srcs, dsts, lsrc, lpk, cnts, db, sb, stage_s, stage_p,
                    cbuf, sem_d):
    c = lax.axis_index("c")
    s = lax.axis_index("s")

    def start_scan(k, b):
        pltpu.async_copy(dsts.at[pl.ds(k * SCH, SCH)], db.at[b], sem_d.at[2 * b])
        pltpu.async_copy(srcs.at[pl.ds(k * SCH, SCH)], sb.at[b], sem_d.at[2 * b + 1])

    def wait_scan(b):
        pltpu.make_async_copy(dsts.at[pl.ds(0, SCH)], db.at[b], sem_d.at[2 * b]).wait()
        pltpu.make_async_copy(srcs.at[pl.ds(0, SCH)], sb.at[b], sem_d.at[2 * b + 1]).wait()

    start_scan(0, 0)

    def chunk(k, ptrs):
        b = k % 2

        @pl.when(k + 1 < NSCH)
        def _():
            start_scan(k + 1, (k + 1) % 2)

        wait_scan(b)

        def vreg(i, ptrs):
            dv = db[b, pl.ds(16 * i, 16)]
            xv = sb[b, pl.ds(16 * i, 16)]
            t = lax.shift_right_logical(dv, 8)
            ev = k * SCH + 16 * i + lax.broadcasted_iota(jnp.int32, (16,), 0)
            pk = lax.shift_left(jnp.bitwise_and(dv, 255), 20) | ev
            new = []
            for j in range(NSLOT):
                ptr = ptrs[j]
                bkt = s + 16 * j
                m = t == bkt
                pc = plsc.all_reduce_population_count(m)[0]
                plsc.store_compressed(stage_s.at[j].at[pl.ds(ptr, 16)], xv, mask=m)
                plsc.store_compressed(stage_p.at[j].at[pl.ds(ptr, 16)], pk, mask=m)
                new.append(jnp.minimum(ptr + pc, SLOTCAP))
            return tuple(new)
        return lax.fori_loop(0, SCH // 16, vreg, ptrs)

    ptrs = lax.fori_loop(0, NSCH, chunk, (0, 0, 0))

    junk_s = jnp.zeros((16,), jnp.int32)
    junk_p = jnp.full((16,), 256 << 20, jnp.int32)
    for j in range(NSLOT):
        ptr = ptrs[j]
        for u in range(8):
            stage_s[j, pl.ds(ptr + 16 * u, 16)] = junk_s
            stage_p[j, pl.ds(ptr + 16 * u, 16)] = junk_p
        nch = (ptr + CHB - 1) // CHB
        cbuf[pl.ds(16 * j, 16)] = jnp.broadcast_to(nch, (16,))

    @pl.when(c == 0)
    def _():
        for j in range(NSLOT):
            r = j * NS + s
            pltpu.sync_copy(stage_s.at[j], lsrc.at[pl.ds(r * CAPAL, CAPAL)])
            pltpu.sync_copy(stage_p.at[j], lpk.at[pl.ds(r * CAPAL, CAPAL)])
            pltpu.sync_copy(cbuf.at[pl.ds(16 * j, 16)], cnts.at[j, s])


_sc_bucket = functools.partial(
    pl.kernel,
    out_type=(
        jax.ShapeDtypeStruct((NSLOT * NS * CAPAL,), jnp.int32),
        jax.ShapeDtypeStruct((NSLOT * NS * CAPAL,), jnp.int32),
        jax.ShapeDtypeStruct((NSLOT, NS, 16), jnp.int32),
    ),
    mesh=plsc.VectorSubcoreMesh(core_axis_name="c", subcore_axis_name="s"),
    scratch_types=[
        pltpu.VMEM((2, SCH), jnp.int32),
        pltpu.VMEM((2, SCH), jnp.int32),
        pltpu.VMEM((NSLOT, CAPAL), jnp.int32),
        pltpu.VMEM((NSLOT, CAPAL), jnp.int32),
        pltpu.VMEM((NSLOT * 16,), jnp.int32),
        pltpu.SemaphoreType.DMA((4,)),
    ],
    compiler_params=pltpu.CompilerParams(use_tc_tiling_on_sc=False,
                                         needs_layout_passes=False),
)(_sc_bucket_body)


NBL = 4


def _sc_edge_body(Dh, x2, lsrc, lpk, cnts, ea2, sflat, out,
                  lsb, lpb, sib, dlb, xg, eag, ob, sv, cbuf, acc,
                  sem_l, sem_g):
    G = Dh // 16
    G2 = (2 * Dh) // 16
    c = lax.axis_index("c")
    s = lax.axis_index("s")

    pltpu.sync_copy(sflat.at[pl.ds(c * Dh, Dh)], sv)
    svs = tuple(sv[pl.ds(16 * j, 16)] for j in range(G))
    for j in range(NSLOT):
        pltpu.sync_copy(cnts.at[j, s], cbuf.at[j])

    for j in range(NSLOT):
        bkt = s + 16 * j
        nch = cbuf[j, pl.ds(0, 16)][0]
        roff = (j * NS + s) * CAPAL

        def zrow(i, _):
            for g in range(G2):
                acc[i, pl.ds(16 * g, 16)] = jnp.zeros((16,), jnp.float32)
            return 0
        lax.fori_loop(0, BR + 8, zrow, 0)

        def start_lists(k, b):
            pltpu.async_copy(lsrc.at[pl.ds(roff + k * CHB, CHB)], lsb.at[b],
                             sem_l.at[2 * b])
            pltpu.async_copy(lpk.at[pl.ds(roff + k * CHB, CHB)], lpb.at[b],
                             sem_l.at[2 * b + 1])

        def wait_lists(b):
            pltpu.make_async_copy(lsrc.at[pl.ds(0, CHB)], lsb.at[b],
                                  sem_l.at[2 * b]).wait()
            pltpu.make_async_copy(lsrc.at[pl.ds(0, CHB)], lpb.at[b],
                                  sem_l.at[2 * b + 1]).wait()

        def start_gathers(k, b):
            pltpu.async_copy(x2.at[sib.at[b]], xg.at[b], sem_g.at[2 * b])
            pltpu.async_copy(ea2.at[pl.ds(c * EAP + roff + k * CHB, CHB)],
                             eag.at[b], sem_g.at[2 * b + 1])

        def wait_gathers(b):
            pltpu.make_async_copy(x2.at[sib.at[b]], xg.at[b],
                                  sem_g.at[2 * b]).wait()
            pltpu.make_async_copy(ea2.at[pl.ds(0, CHB)], eag.at[b],
                                  sem_g.at[2 * b + 1]).wait()

        def unpack(b):
            for i in range(CHB // 16):
                sl = pl.ds(16 * i, 16)
                sib[b, sl] = lsb[b, sl] + c * N
                dlb[b, sl] = lax.shift_right_logical(lpb[b, sl], 20)

        for p in range(4):
            @pl.when(p < nch)
            def _():
                start_lists(p, p)

        def stage2(k, b):
            wait_lists(b)
            unpack(b)
            start_gathers(k, b)

        for p in range(2):
            @pl.when(p < nch)
            def _():
                stage2(p, p)

        def compute(kb):
            def rowA(q, _):
                for r in range(4):
                    i = 4 * q + r
                    for g in range(G):
                        xv = xg[kb, i, pl.ds(16 * g, 16)]
                        ev = eag[kb, i, pl.ds(16 * g, 16)]
                        msg = jnp.maximum(xv + ev + EPS, EPS)
                        w = jnp.exp(msg - svs[g])
                        ob[i, pl.ds(16 * g, 16)] = w
                        ob[i, pl.ds(Dh + 16 * g, 16)] = msg * w
                return 0
            lax.fori_loop(0, CHB // 4, rowA, 0)

            def rowB(q, _):
                dlv = dlb[kb, pl.ds(16 * q, 16)]
                for r in range(16):
                    dl = dlv[r]
                    i = 16 * q + r
                    for g in range(G2):
                        plsc.addupdate(acc.at[dl].at[pl.ds(16 * g, 16)],
                                       ob[i, pl.ds(16 * g, 16)])
                return 0
            lax.fori_loop(0, CHB // 16, rowB, 0)

        def chunk(k, _):
            @pl.when(k + 4 < nch)
            def _():
                start_lists(k + 4, (k + 4) % NBL)

            @pl.when(k + 2 < nch)
            def _():
                stage2(k + 2, (k + 2) % NBL)

            wait_gathers(k % NBL)
            compute(k % NBL)
            return 0
        lax.fori_loop(0, nch, chunk, 0)

        @pl.when(bkt < NBKT)
        def _():
            pltpu.sync_copy(acc.at[pl.ds(0, BR)],
                            out.at[pl.ds(c * NPAD + bkt * BR, BR)])


def _make_sc_edge(Dh):
    mesh = plsc.VectorSubcoreMesh(core_axis_name="c", subcore_axis_name="s")
    return functools.partial(
        pl.kernel,
        out_type=jax.ShapeDtypeStruct((NC * NPAD, 2 * Dh), jnp.float32),
        mesh=mesh,
        scratch_types=[
            pltpu.VMEM((NBL, CHB), jnp.int32),
            pltpu.VMEM((NBL, CHB), jnp.int32),
            pltpu.VMEM((NBL, CHB), jnp.int32),
            pltpu.VMEM((NBL, CHB), jnp.int32),
            pltpu.VMEM((NBL, CHB, Dh), jnp.float32),
            pltpu.VMEM((NBL, CHB, Dh), jnp.float32),
            pltpu.VMEM((CHB, 2 * Dh), jnp.float32),
            pltpu.VMEM((Dh,), jnp.float32),
            pltpu.VMEM((NSLOT, 16), jnp.int32),
            pltpu.VMEM((BR + 8, 2 * Dh), jnp.float32),
            pltpu.SemaphoreType.DMA((2 * NBL,)),
            pltpu.SemaphoreType.DMA((2 * NBL,)),
        ],
        compiler_params=pltpu.CompilerParams(use_tc_tiling_on_sc=False),
    )(functools.partial(_sc_edge_body, Dh))


_sc_edge_64 = _make_sc_edge(64)
_sc_edge_32 = _make_sc_edge(32)


TE = 4000


def _ea_body(attr, Wc, bc, o1, o2, o3):
    ea = jnp.dot(attr[...], Wc[...], preferred_element_type=jnp.float32) + bc[...]
    o1[0] = ea[:, 0:64]
    o1[1] = ea[:, 64:128]
    o2[0] = ea[:, 128:160]
    o2[1] = ea[:, 160:192]
    o3[0] = ea[:, 192:224]
    o3[1] = ea[:, 224:256]


def _ea_all(edge_attr, Wc, bc):
    return pl.pallas_call(
        _ea_body,
        grid=(EAP // TEB,),
        in_specs=[
            pl.BlockSpec((TEB, 16), lambda i: (i, 0)),
            pl.BlockSpec((16, 256), lambda i: (0, 0)),
            pl.BlockSpec((1, 256), lambda i: (0, 0)),
        ],
        out_specs=[
            pl.BlockSpec((2, TEB, 64), lambda i: (0, i, 0)),
            pl.BlockSpec((2, TEB, 32), lambda i: (0, i, 0)),
            pl.BlockSpec((2, TEB, 32), lambda i: (0, i, 0)),
        ],
        out_shape=[
            jax.ShapeDtypeStruct((2, EAP, 64), jnp.float32),
            jax.ShapeDtypeStruct((2, EAP, 32), jnp.float32),
            jax.ShapeDtypeStruct((2, EAP, 32), jnp.float32),
        ],
    )(edge_attr, Wc, bc)


TN = 1000
NGRID = N // TN


def _aggr_mlp1_body(acc, xs, W1, b1, h_out, sh_out, sh2_out, sh_s, sh2_s):
    i = pl.program_id(0)
    accb = acc[...]
    Dh = accb.shape[2] // 2
    den = jnp.concatenate([accb[0, :, 0:Dh], accb[1, :, 0:Dh]], axis=1)
    num = jnp.concatenate([accb[0, :, Dh:], accb[1, :, Dh:]], axis=1)
    aggr = num / jnp.maximum(den, 1e-38)
    xsb = xs[...]
    xb = jnp.concatenate([xsb[0], xsb[1]], axis=1)
    out = xb + aggr
    h = jnp.dot(out, W1[...], preferred_element_type=jnp.float32) + b1[...]
    h_out[...] = h

    @pl.when(i == 0)
    def _():
        sh_s[...] = jnp.zeros_like(sh_s)
        sh2_s[...] = jnp.zeros_like(sh2_s)

    sh_s[...] += jnp.sum(h, axis=0, keepdims=True)
    sh2_s[...] += jnp.sum(h * h, axis=0, keepdims=True)

    @pl.when(i == NGRID - 1)
    def _():
        sh_out[...] = sh_s[...]
        sh2_out[...] = sh2_s[...]


def _aggr_mlp1(acc3, xsplit, W1, b1):
    D = W1.shape[0]
    H = W1.shape[1]
    return pl.pallas_call(
        _aggr_mlp1_body,
        grid=(NGRID,),
        in_specs=[
            pl.BlockSpec((2, TN, D), lambda i: (0, i, 0)),
            pl.BlockSpec((2, TN, D // 2), lambda i: (0, i, 0)),
            pl.BlockSpec((D, H), lambda i: (0, 0)),
            pl.BlockSpec((1, H), lambda i: (0, 0)),
        ],
        out_specs=[
            pl.BlockSpec((TN, H), lambda i: (i, 0)),
            pl.BlockSpec((1, H), lambda i: (0, 0)),
            pl.BlockSpec((1, H), lambda i: (0, 0)),
        ],
        out_shape=[
            jax.ShapeDtypeStruct((N, H), jnp.float32),
            jax.ShapeDtypeStruct((1, H), jnp.float32),
            jax.ShapeDtypeStruct((1, H), jnp.float32),
        ],
        scratch_shapes=[
            pltpu.VMEM((1, H), jnp.float32),
            pltpu.VMEM((1, H), jnp.float32),
        ],
    )(acc3, xsplit, W1, b1)


def _bn_mlp2_body(h, sh, sh2, g, bt, W2, b2, y_out, xmax_out, xmax_s):
    i = pl.program_id(0)
    mu = sh[...] / N
    var = sh2[...] / N - mu * mu
    hn = (h[...] - mu) * lax.rsqrt(var + 1e-5) * g[...] + bt[...]
    hn = jnp.maximum(hn, 0.0)
    y = jnp.dot(hn, W2[...], preferred_element_type=jnp.float32) + b2[...]
    y = jnp.maximum(y, 0.0)
    Dh = y.shape[1] // 2
    y_out[0] = y[:, 0:Dh]
    y_out[1] = y[:, Dh:]

    @pl.when(i == 0)
    def _():
        xmax_s[...] = jnp.full_like(xmax_s, -jnp.inf)

    xmax_s[...] = jnp.maximum(xmax_s[...], jnp.max(y, axis=0, keepdims=True))

    @pl.when(i == NGRID - 1)
    def _():
        xmax_out[...] = xmax_s[...]


def _bn_mlp2(h, sh, sh2, g, bt, W2, b2):
    H = W2.shape[0]
    Do = W2.shape[1]
    return pl.pallas_call(
        _bn_mlp2_body,
        grid=(NGRID,),
        in_specs=[
            pl.BlockSpec((TN, H), lambda i: (i, 0)),
            pl.BlockSpec((1, H), lambda i: (0, 0)),
            pl.BlockSpec((1, H), lambda i: (0, 0)),
            pl.BlockSpec((1, H), lambda i: (0, 0)),
            pl.BlockSpec((1, H), lambda i: (0, 0)),
            pl.BlockSpec((H, Do), lambda i: (0, 0)),
            pl.BlockSpec((1, Do), lambda i: (0, 0)),
        ],
        out_specs=[
            pl.BlockSpec((2, TN, Do // 2), lambda i: (0, i, 0)),
            pl.BlockSpec((1, Do), lambda i: (0, 0)),
        ],
        out_shape=[
            jax.ShapeDtypeStruct((2, N, Do // 2), jnp.float32),
            jax.ShapeDtypeStruct((1, Do), jnp.float32),
        ],
        scratch_shapes=[pltpu.VMEM((1, Do), jnp.float32)],
    )(h, sh, sh2, g, bt, W2, b2)


def _pool_head_body(hsplit, batch3, d1W, d1b, d2W, d2b, out, pool_s, cnt_s):
    i = pl.program_id(0)
    b = batch3[...].reshape(1, TN)
    gid = lax.broadcasted_iota(jnp.int32, (N_GRAPHS, TN), 0)
    onehot = jnp.where(gid == b, 1.0, 0.0).astype(jnp.float32)
    hb = hsplit[...]
    hcat = jnp.concatenate([hb[0], hb[1]], axis=1)

    @pl.when(i == 0)
    def _():
        pool_s[...] = jnp.zeros_like(pool_s)
        cnt_s[...] = jnp.zeros_like(cnt_s)

    pool_s[...] += jnp.dot(onehot, hcat, preferred_element_type=jnp.float32)
    cnt_s[...] += jnp.sum(onehot, axis=1, keepdims=True)

    @pl.when(i == NGRID - 1)
    def _():
        mean = pool_s[...] / jnp.maximum(cnt_s[...], 1.0)
        z = jnp.dot(mean, d1W[...], preferred_element_type=jnp.float32) + d1b[...]
        z = jnp.dot(z, d2W[...], preferred_element_type=jnp.float32) + d2b[...]
        m = jnp.max(z, axis=-1, keepdims=True)
        lse = m + jnp.log(jnp.sum(jnp.exp(z - m), axis=-1, keepdims=True))
        out[...] = z - lse


def _pool_head(hsplit, batch3, d1W, d1b, d2W, d2b):
    return pl.pallas_call(
        _pool_head_body,
        grid=(NGRID,),
        in_specs=[
            pl.BlockSpec((2, TN, 64), lambda i: (0, i, 0)),
            pl.BlockSpec((1, 1, TN), lambda i: (i, 0, 0)),
            pl.BlockSpec((128, 64), lambda i: (0, 0)),
            pl.BlockSpec((1, 64), lambda i: (0, 0)),
            pl.BlockSpec((64, 10), lambda i: (0, 0)),
            pl.BlockSpec((1, 10), lambda i: (0, 0)),
        ],
        out_specs=pl.BlockSpec((N_GRAPHS, 10), lambda i: (0, 0)),
        out_shape=jax.ShapeDtypeStruct((N_GRAPHS, 10), jnp.float32),
        scratch_shapes=[
            pltpu.VMEM((N_GRAPHS, 128), jnp.float32),
            pltpu.VMEM((N_GRAPHS, 1), jnp.float32),
        ],
    )(hsplit, batch3, d1W, d1b, d2W, d2b)


def _layer(xsplit, xmax, lsrc, lpk, cnts, ea2, colmax, We, be, W1, b1, g, bt, W2, b2):
    D = W1.shape[0]
    Dh = D // 2
    eabound = jnp.abs(We).T @ colmax + be
    S = jnp.maximum(xmax + eabound, 0.0) + EPS
    sc = _sc_edge_64 if Dh == 64 else _sc_edge_32
    acc = sc(xsplit.reshape(2 * N, Dh), lsrc, lpk, cnts, ea2.reshape(2 * EAP, Dh), S)
    acc3 = acc.reshape(2, NPAD, 2 * Dh)[:, :N, :]
    h, sh, sh2 = _aggr_mlp1(acc3, xsplit, W1, b1[None, :])
    return _bn_mlp2(h, sh, sh2, g[None, :], bt[None, :], W2, b2[None, :])


def kernel(x, edge_index, edge_attr, batch,
           c1_We, c1_be, c1_W1, c1_b1, c1_g, c1_bt, c1_W2, c1_b2,
           c2_We, c2_be, c2_W1, c2_b1, c2_g, c2_bt, c2_W2, c2_b2,
           c3_We, c3_be, c3_W1, c3_b1, c3_g, c3_bt, c3_W2, c3_b2,
           d1_W, d1_b, d2_W, d2_b):
    srcs = edge_index[0]
    dsts = edge_index[1]
    lsrc, lpk, cnts = _sc_bucket(srcs, dsts)
    perm = jnp.bitwise_and(lpk, (1 << 20) - 1)
    attr_perm = jnp.take(edge_attr, perm, axis=0)
    Wc = jnp.concatenate([c1_We, c2_We, c3_We], axis=1)
    bc = jnp.concatenate([c1_be, c2_be, c3_be])[None, :]
    ea1, ea2, ea3 = _ea_all(attr_perm, Wc, bc)
    colmax = jnp.max(jnp.abs(edge_attr), axis=0)

    x1 = x.reshape(N, 2, 64).transpose(1, 0, 2)
    xmax1 = jnp.max(x, axis=0)
    x2s, xmax2 = _layer(x1, xmax1, lsrc, lpk, cnts, ea1, colmax,
                        c1_We, c1_be, c1_W1, c1_b1, c1_g, c1_bt, c1_W2, c1_b2)
    x3s, xmax3 = _layer(x2s, xmax2.reshape(-1), lsrc, lpk, cnts, ea2, colmax,
                        c2_We, c2_be, c2_W1, c2_b1, c2_g, c2_bt, c2_W2, c2_b2)
    x4s, _ = _layer(x3s, xmax3.reshape(-1), lsrc, lpk, cnts, ea3, colmax,
                    c3_We, c3_be, c3_W1, c3_b1, c3_g, c3_bt, c3_W2, c3_b2)

    batch3 = batch.reshape(NGRID, 1, TN)
    return _pool_head(x4s, batch3, d1_W, d1_b[None, :], d2_W, d2_b[None, :])


# speedup vs baseline: 1.9720x; 1.9720x over previous
import functools

import jax
import jax.numpy as jnp
from jax import lax
from jax.experimental import pallas as pl
from jax.experimental.pallas import tpu as pltpu
from jax.experimental.pallas import tpu_sc as plsc

N = 10000
E = 320000
N_GRAPHS = 64
EPS = 1e-7

NC = 2
NS = 16
EP = E // NS


def _sc_edge_body(Dh, CH, NB, x2, srcs2, dsts, ea2, sflat, out, isrc, idst,
                  xg, eab, ob, sv, acc, sem_i, sem_e, sem_g, sem_s):
    NCHUNK = EP // CH
    NSUP = NCHUNK // NB
    NZ = N // CH
    NZT = (NZ + NS - 1) // NS
    G = Dh // 16
    G2 = (2 * Dh) // 16
    c = lax.axis_index("c")
    s = lax.axis_index("s")

    def zrow(i, _):
        for j in range(G2):
            ob[0, i, pl.ds(16 * j, 16)] = jnp.zeros((16,), jnp.float32)
        return 0
    lax.fori_loop(0, CH, zrow, 0)
    for t in range(NZT):
        zi = s + NS * t

        @pl.when(zi < NZ)
        def _():
            pltpu.sync_copy(ob.at[0], acc.at[pl.ds(zi * CH, CH)])

    pltpu.sync_copy(sflat.at[pl.ds(c * Dh, Dh)], sv)
    svs = tuple(sv[pl.ds(16 * j, 16)] for j in range(G))
    plsc.subcore_barrier()

    def start_idx(k, b):
        base = s * EP + k * CH
        pltpu.async_copy(srcs2.at[pl.ds(c * E + base, CH)], isrc.at[b], sem_i.at[b])
        pltpu.async_copy(dsts.at[pl.ds(base, CH)], idst.at[b], sem_i.at[b])
        pltpu.async_copy(ea2.at[pl.ds(c * E + base, CH)], eab.at[b], sem_e.at[b])

    def wait_idx(b):
        pltpu.make_async_copy(dsts.at[pl.ds(0, CH)], isrc.at[b], sem_i.at[b]).wait()
        pltpu.make_async_copy(dsts.at[pl.ds(0, CH)], idst.at[b], sem_i.at[b]).wait()

    def start_gather(b):
        pltpu.async_copy(x2.at[isrc.at[b]], xg.at[b], sem_g.at[b])

    def wait_gather_ea(b):
        pltpu.make_async_copy(x2.at[isrc.at[b]], xg.at[b], sem_g.at[b]).wait()
        pltpu.make_async_copy(ea2.at[pl.ds(0, CH)], eab.at[b], sem_e.at[b]).wait()

    def start_scatter(b):
        pltpu.async_copy(ob.at[b], acc.at[idst.at[b]], sem_s.at[b], add=True)

    def wait_scatter(b):
        pltpu.make_async_copy(ob.at[b], acc.at[idst.at[b]], sem_s.at[b]).wait()

    start_idx(0, 0)
    start_idx(1, 1)
    wait_idx(0)
    start_gather(0)

    def super_chunk(k5, carry):
        svs = carry
        for b in range(NB):
            k = k5 * NB + b

            @pl.when(k >= 2)
            def _():
                wait_scatter((b - 2) % NB)

            @pl.when(k + 2 < NCHUNK)
            def _():
                start_idx(k + 2, (b + 2) % NB)

            @pl.when(k + 1 < NCHUNK)
            def _():
                wait_idx((b + 1) % NB)
                start_gather((b + 1) % NB)

            wait_gather_ea(b)

            def row(i, _):
                for r in range(2):
                    for j in range(G):
                        xv = xg[b, 2 * i + r, pl.ds(16 * j, 16)]
                        ev = eab[b, 2 * i + r, pl.ds(16 * j, 16)]
                        msg = jnp.maximum(xv + ev + EPS, EPS)
                        w = jnp.exp(msg - svs[j])
                        ob[b, 2 * i + r, pl.ds(16 * j, 16)] = w
                        ob[b, 2 * i + r, pl.ds(Dh + 16 * j, 16)] = msg * w
                return 0
            lax.fori_loop(0, CH // 2, row, 0)
            start_scatter(b)
        return svs
    lax.fori_loop(0, NSUP, super_chunk, svs)
    wait_scatter((NCHUNK - 2) % NB)
    wait_scatter((NCHUNK - 1) % NB)

    plsc.subcore_barrier()
    for t in range(NZT):
        zi = s + NS * t

        @pl.when(zi < NZ)
        def _():
            pltpu.sync_copy(acc.at[pl.ds(zi * CH, CH)], ob.at[0])
            pltpu.sync_copy(ob.at[0], out.at[pl.ds(c * N + zi * CH, CH)])


def _make_sc_edge(Dh, CH, NB):
    mesh = plsc.VectorSubcoreMesh(core_axis_name="c", subcore_axis_name="s")
    return functools.partial(
        pl.kernel,
        out_type=jax.ShapeDtypeStruct((NC * N, 2 * Dh), jnp.float32),
        mesh=mesh,
        scratch_types=[
            pltpu.VMEM((NB, CH), jnp.int32),
            pltpu.VMEM((NB, CH), jnp.int32),
            pltpu.VMEM((NB, CH, Dh), jnp.float32),
            pltpu.VMEM((NB, CH, Dh), jnp.float32),
            pltpu.VMEM((NB, CH, 2 * Dh), jnp.float32),
            pltpu.VMEM((Dh,), jnp.float32),
            pltpu.VMEM_SHARED((N, 2 * Dh), jnp.float32),
            pltpu.SemaphoreType.DMA((NB,)),
            pltpu.SemaphoreType.DMA((NB,)),
            pltpu.SemaphoreType.DMA((NB,)),
            pltpu.SemaphoreType.DMA((NB,)),
        ],
        compiler_params=pltpu.CompilerParams(use_tc_tiling_on_sc=False),
    )(functools.partial(_sc_edge_body, Dh, CH, NB))


_sc_edge_64 = _make_sc_edge(64, 40, 4)
_sc_edge_32 = _make_sc_edge(32, 80, 5)


TE = 4000


def _ea_body(attr, Wc, bc, o1, o2, o3):
    ea = jnp.dot(attr[...], Wc[...], preferred_element_type=jnp.float32) + bc[...]
    o1[0] = ea[:, 0:64]
    o1[1] = ea[:, 64:128]
    o2[0] = ea[:, 128:160]
    o2[1] = ea[:, 160:192]
    o3[0] = ea[:, 192:224]
    o3[1] = ea[:, 224:256]


def _ea_all(edge_attr, Wc, bc):
    return pl.pallas_call(
        _ea_body,
        grid=(E // TE,),
        in_specs=[
            pl.BlockSpec((TE, 16), lambda i: (i, 0)),
            pl.BlockSpec((16, 256), lambda i: (0, 0)),
            pl.BlockSpec((1, 256), lambda i: (0, 0)),
        ],
        out_specs=[
            pl.BlockSpec((2, TE, 64), lambda i: (0, i, 0)),
            pl.BlockSpec((2, TE, 32), lambda i: (0, i, 0)),
            pl.BlockSpec((2, TE, 32), lambda i: (0, i, 0)),
        ],
        out_shape=[
            jax.ShapeDtypeStruct((2, E, 64), jnp.float32),
            jax.ShapeDtypeStruct((2, E, 32), jnp.float32),
            jax.ShapeDtypeStruct((2, E, 32), jnp.float32),
        ],
    )(edge_attr, Wc, bc)


TN = 1000
NGRID = N // TN


def _aggr_mlp1_body(acc, xs, W1, b1, h_out, sh_out, sh2_out, sh_s, sh2_s):
    i = pl.program_id(0)
    accb = acc[...]
    Dh = accb.shape[2] // 2
    den = jnp.concatenate([accb[0, :, 0:Dh], accb[1, :, 0:Dh]], axis=1)
    num = jnp.concatenate([accb[0, :, Dh:], accb[1, :, Dh:]], axis=1)
    aggr = num / jnp.maximum(den, 1e-38)
    xsb = xs[...]
    xb = jnp.concatenate([xsb[0], xsb[1]], axis=1)
    out = xb + aggr
    h = jnp.dot(out, W1[...], preferred_element_type=jnp.float32) + b1[...]
    h_out[...] = h

    @pl.when(i == 0)
    def _():
        sh_s[...] = jnp.zeros_like(sh_s)
        sh2_s[...] = jnp.zeros_like(sh2_s)

    sh_s[...] += jnp.sum(h, axis=0, keepdims=True)
    sh2_s[...] += jnp.sum(h * h, axis=0, keepdims=True)

    @pl.when(i == NGRID - 1)
    def _():
        sh_out[...] = sh_s[...]
        sh2_out[...] = sh2_s[...]


def _aggr_mlp1(acc3, xsplit, W1, b1):
    D = W1.shape[0]
    H = W1.shape[1]
    return pl.pallas_call(
        _aggr_mlp1_body,
        grid=(NGRID,),
        in_specs=[
            pl.BlockSpec((2, TN, D), lambda i: (0, i, 0)),
            pl.BlockSpec((2, TN, D // 2), lambda i: (0, i, 0)),
            pl.BlockSpec((D, H), lambda i: (0, 0)),
            pl.BlockSpec((1, H), lambda i: (0, 0)),
        ],
        out_specs=[
            pl.BlockSpec((TN, H), lambda i: (i, 0)),
            pl.BlockSpec((1, H), lambda i: (0, 0)),
            pl.BlockSpec((1, H), lambda i: (0, 0)),
        ],
        out_shape=[
            jax.ShapeDtypeStruct((N, H), jnp.float32),
            jax.ShapeDtypeStruct((1, H), jnp.float32),
            jax.ShapeDtypeStruct((1, H), jnp.float32),
        ],
        scratch_shapes=[
            pltpu.VMEM((1, H), jnp.float32),
            pltpu.VMEM((1, H), jnp.float32),
        ],
    )(acc3, xsplit, W1, b1)


def _bn_mlp2_body(h, sh, sh2, g, bt, W2, b2, y_out, xmax_out, xmax_s):
    i = pl.program_id(0)
    mu = sh[...] / N
    var = sh2[...] / N - mu * mu
    hn = (h[...] - mu) * lax.rsqrt(var + 1e-5) * g[...] + bt[...]
    hn = jnp.maximum(hn, 0.0)
    y = jnp.dot(hn, W2[...], preferred_element_type=jnp.float32) + b2[...]
    y = jnp.maximum(y, 0.0)
    Dh = y.shape[1] // 2
    y_out[0] = y[:, 0:Dh]
    y_out[1] = y[:, Dh:]

    @pl.when(i == 0)
    def _():
        xmax_s[...] = jnp.full_like(xmax_s, -jnp.inf)

    xmax_s[...] = jnp.maximum(xmax_s[...], jnp.max(y, axis=0, keepdims=True))

    @pl.when(i == NGRID - 1)
    def _():
        xmax_out[...] = xmax_s[...]


def _bn_mlp2(h, sh, sh2, g, bt, W2, b2):
    H = W2.shape[0]
    Do = W2.shape[1]
    return pl.pallas_call(
        _bn_mlp2_body,
        grid=(NGRID,),
        in_specs=[
            pl.BlockSpec((TN, H), lambda i: (i, 0)),
            pl.BlockSpec((1, H), lambda i: (0, 0)),
            pl.BlockSpec((1, H), lambda i: (0, 0)),
            pl.BlockSpec((1, H), lambda i: (0, 0)),
            pl.BlockSpec((1, H), lambda i: (0, 0)),
            pl.BlockSpec((H, Do), lambda i: (0, 0)),
            pl.BlockSpec((1, Do), lambda i: (0, 0)),
        ],
        out_specs=[
            pl.BlockSpec((2, TN, Do // 2), lambda i: (0, i, 0)),
            pl.BlockSpec((1, Do), lambda i: (0, 0)),
        ],
        out_shape=[
            jax.ShapeDtypeStruct((2, N, Do // 2), jnp.float32),
            jax.ShapeDtypeStruct((1, Do), jnp.float32),
        ],
        scratch_shapes=[pltpu.VMEM((1, Do), jnp.float32)],
    )(h, sh, sh2, g, bt, W2, b2)


def _pool_head_body(hsplit, batch3, d1W, d1b, d2W, d2b, out, pool_s, cnt_s):
    i = pl.program_id(0)
    b = batch3[...].reshape(1, TN)
    gid = lax.broadcasted_iota(jnp.int32, (N_GRAPHS, TN), 0)
    onehot = jnp.where(gid == b, 1.0, 0.0).astype(jnp.float32)
    hb = hsplit[...]
    hcat = jnp.concatenate([hb[0], hb[1]], axis=1)

    @pl.when(i == 0)
    def _():
        pool_s[...] = jnp.zeros_like(pool_s)
        cnt_s[...] = jnp.zeros_like(cnt_s)

    pool_s[...] += jnp.dot(onehot, hcat, preferred_element_type=jnp.float32)
    cnt_s[...] += jnp.sum(onehot, axis=1, keepdims=True)

    @pl.when(i == NGRID - 1)
    def _():
        mean = pool_s[...] / jnp.maximum(cnt_s[...], 1.0)
        z = jnp.dot(mean, d1W[...], preferred_element_type=jnp.float32) + d1b[...]
        z = jnp.dot(z, d2W[...], preferred_element_type=jnp.float32) + d2b[...]
        m = jnp.max(z, axis=-1, keepdims=True)
        lse = m + jnp.log(jnp.sum(jnp.exp(z - m), axis=-1, keepdims=True))
        out[...] = z - lse


def _pool_head(hsplit, batch3, d1W, d1b, d2W, d2b):
    return pl.pallas_call(
        _pool_head_body,
        grid=(NGRID,),
        in_specs=[
            pl.BlockSpec((2, TN, 64), lambda i: (0, i, 0)),
            pl.BlockSpec((1, 1, TN), lambda i: (i, 0, 0)),
            pl.BlockSpec((128, 64), lambda i: (0, 0)),
            pl.BlockSpec((1, 64), lambda i: (0, 0)),
            pl.BlockSpec((64, 10), lambda i: (0, 0)),
            pl.BlockSpec((1, 10), lambda i: (0, 0)),
        ],
        out_specs=pl.BlockSpec((N_GRAPHS, 10), lambda i: (0, 0)),
        out_shape=jax.ShapeDtypeStruct((N_GRAPHS, 10), jnp.float32),
        scratch_shapes=[
            pltpu.VMEM((N_GRAPHS, 128), jnp.float32),
            pltpu.VMEM((N_GRAPHS, 1), jnp.float32),
        ],
    )(hsplit, batch3, d1W, d1b, d2W, d2b)


def _layer(xsplit, xmax, srcs2, dsts, ea2, colmax, We, be, W1, b1, g, bt, W2, b2):
    D = W1.shape[0]
    Dh = D // 2
    eabound = jnp.abs(We).T @ colmax + be
    S = jnp.maximum(xmax + eabound, 0.0) + EPS
    sc = _sc_edge_64 if Dh == 64 else _sc_edge_32
    acc = sc(xsplit.reshape(2 * N, Dh), srcs2, dsts, ea2.reshape(2 * E, Dh), S)
    acc3 = acc.reshape(2, N, 2 * Dh)
    h, sh, sh2 = _aggr_mlp1(acc3, xsplit, W1, b1[None, :])
    return _bn_mlp2(h, sh, sh2, g[None, :], bt[None, :], W2, b2[None, :])


def kernel(x, edge_index, edge_attr, batch,
           c1_We, c1_be, c1_W1, c1_b1, c1_g, c1_bt, c1_W2, c1_b2,
           c2_We, c2_be, c2_W1, c2_b1, c2_g, c2_bt, c2_W2, c2_b2,
           c3_We, c3_be, c3_W1, c3_b1, c3_g, c3_bt, c3_W2, c3_b2,
           d1_W, d1_b, d2_W, d2_b):
    srcs = edge_index[0]
    dsts = edge_index[1]
    srcs2 = jnp.concatenate([srcs, srcs + N])
    Wc = jnp.concatenate([c1_We, c2_We, c3_We], axis=1)
    bc = jnp.concatenate([c1_be, c2_be, c3_be])[None, :]
    ea1, ea2, ea3 = _ea_all(edge_attr, Wc, bc)
    colmax = jnp.max(jnp.abs(edge_attr), axis=0)

    x1 = x.reshape(N, 2, 64).transpose(1, 0, 2)
    xmax1 = jnp.max(x, axis=0)
    x2s, xmax2 = _layer(x1, xmax1, srcs2, dsts, ea1, colmax,
                        c1_We, c1_be, c1_W1, c1_b1, c1_g, c1_bt, c1_W2, c1_b2)
    x3s, xmax3 = _layer(x2s, xmax2.reshape(-1), srcs2, dsts, ea2, colmax,
                        c2_We, c2_be, c2_W1, c2_b1, c2_g, c2_bt, c2_W2, c2_b2)
    x4s, _ = _layer(x3s, xmax3.reshape(-1), srcs2, dsts, ea3, colmax,
                    c3_We, c3_be, c3_W1, c3_b1, c3_g, c3_bt, c3_W2, c3_b2)

    batch3 = batch.reshape(NGRID, 1, TN)
    return _pool_head(x4s, batch3, d1_W, d1_b[None, :], d2_W, d2_b[None, :])
